# baseline (speedup 1.0000x reference)
import jax
import jax.numpy as jnp
from jax import lax
from jax.experimental import pallas as pl
from jax.experimental.pallas import tpu as pltpu
from jax.experimental.pallas import tpu_sc as plsc

N = 10000
E = 320000
D = 128
H = 4

NC = 2
NS = 16
NW = NC * NS
CH = 128
TR = E // CH
RPW = 80


def _worker_rows(wid):
    nch = jnp.where(wid == NW - 1, TR - (NW - 1) * RPW, RPW)
    return wid * RPW, nch


def _ei_chunk(ei_hbm, row):
    return ei_hbm.at[pl.ds(0, 2), pl.ds(row * CH, CH)]


def _deg_body(dst_hbm, ones_hbm, zeros_hbm, out_hbm,
              deg_sp, dstv, onesv, zv, dsems):
    c = lax.axis_index("c")
    s = lax.axis_index("s")
    wid = c * NS + s
    base, nch = _worker_rows(wid)

    @pl.when(wid < NW - 1)
    def _():
        pltpu.sync_copy(dst_hbm.at[pl.ds(base, RPW)], dstv)

    @pl.when(wid == NW - 1)
    def _():
        pltpu.sync_copy(dst_hbm.at[pl.ds((NW - 1) * RPW, 20)],
                        dstv.at[pl.ds(0, 20)])
    pltpu.sync_copy(ones_hbm, onesv)
    @pl.when(s < 10)
    def _():
        pltpu.sync_copy(zeros_hbm, zv)
        pltpu.sync_copy(zv, deg_sp.at[pl.ds(s * 1000, 1000)])
    plsc.subcore_barrier()

    def body(g, carry):
        for u in range(4):
            j = g * 4 + u

            @pl.when(j >= 4)
            def _():
                pltpu.make_async_copy(onesv, deg_sp.at[dstv.at[0]],
                                      dsems[u]).wait()
            pltpu.async_copy(onesv, deg_sp.at[dstv.at[j]], dsems[u],
                             add=True)
        return carry

    lax.fori_loop(0, nch // 4, body, 0)
    for u in range(4):
        pltpu.make_async_copy(onesv, deg_sp.at[dstv.at[0]], dsems[u]).wait()
    plsc.subcore_barrier()

    @pl.when(s < 10)
    def _():
        pltpu.sync_copy(deg_sp.at[pl.ds(s * 1000, 1000)], zv)
        pltpu.sync_copy(zv, out_hbm.at[c, s])


def _agg_body(y_hbm, ei_hbm, zeros_hbm, out_hbm,
              agg_sp, sring, bufs, isems, gsems, ssems):
    c = lax.axis_index("c")
    s = lax.axis_index("s")
    wid = c * NS + s
    base, nch = _worker_rows(wid)
    pltpu.sync_copy(zeros_hbm, bufs[0])

    @pl.when(s < 15)
    def _():
        for k in range(5):
            pltpu.sync_copy(bufs[0],
                            agg_sp.at[pl.ds(s * 640 + k * 128, 128)])

    @pl.when(s == 15)
    def _():
        for k in range(3):
            pltpu.sync_copy(bufs[0],
                            agg_sp.at[pl.ds(9600 + k * 128, 128)])
        pltpu.sync_copy(bufs[0].at[pl.ds(0, 16)],
                        agg_sp.at[pl.ds(9984, 16)])
    plsc.subcore_barrier()

    for r in range(3):
        pltpu.async_copy(_ei_chunk(ei_hbm, base + r), sring[r], isems[r])
    pltpu.make_async_copy(_ei_chunk(ei_hbm, base), sring[0], isems[0]).wait()
    pltpu.async_copy(y_hbm.at[sring[0].at[0]], bufs[0], gsems[0])

    def group(g, carry):
        for u in range(4):
            j = g * 4 + u
            rn = (u + 1) % 4
            bn = (u + 1) % 2

            @pl.when(j + 1 < nch)
            def _():
                @pl.when(j >= 1)
                def _():
                    pltpu.make_async_copy(bufs[bn],
                                          agg_sp.at[sring[0].at[1]],
                                          ssems[bn]).wait()
                pltpu.make_async_copy(_ei_chunk(ei_hbm, base), sring[rn],
                                      isems[rn]).wait()
                pltpu.async_copy(y_hbm.at[sring[rn].at[0]], bufs[bn],
                                 gsems[bn])

            @pl.when(j + 3 < nch)
            def _():
                pltpu.async_copy(_ei_chunk(ei_hbm, base + j + 3),
                                 sring[(u + 3) % 4], isems[(u + 3) % 4])

            pltpu.make_async_copy(y_hbm.at[sring[u].at[0]], bufs[u % 2],
                                  gsems[u % 2]).wait()
            pltpu.async_copy(bufs[u % 2], agg_sp.at[sring[u].at[1]],
                             ssems[u % 2], add=True)
        return carry

    lax.fori_loop(0, nch // 4, group, 0)
    for b in range(2):
        pltpu.make_async_copy(bufs[b], agg_sp.at[sring[0].at[1]],
                              ssems[b]).wait()
    plsc.subcore_barrier()

    @pl.when(s < 15)
    def _():
        pltpu.sync_copy(agg_sp.at[pl.ds(s * 640, 640)],
                        out_hbm.at[c, pl.ds(s * 640, 640)])

    @pl.when(s == 15)
    def _():
        pltpu.sync_copy(agg_sp.at[pl.ds(9600, 400)],
                        out_hbm.at[c, pl.ds(9600, 400)])


def _scale_body(x_ref, dp_ref, y_ref, dis_ref):
    deg = dp_ref[0] + dp_ref[1] + 1.0
    dis = lax.rsqrt(deg)
    y_ref[...] = x_ref[...] * dis
    dis_ref[...] = dis


def _head_body(ap_ref, y_ref, dis_ref, w_ref, b_ref, o_ref):
    agg = (ap_ref[0] + ap_ref[1] + y_ref[...]) * dis_ref[...]
    o_ref[...] = (
        jnp.dot(agg, w_ref[...], preferred_element_type=jnp.float32)
        + b_ref[...]
    )


def _sc_mesh():
    return plsc.VectorSubcoreMesh(core_axis_name="c", subcore_axis_name="s")


@jax.jit
def kernel(x, edge_index, W, b):
    dst = edge_index[1].reshape(TR, CH)
    deg_part = pl.kernel(
        _deg_body,
        out_type=jax.ShapeDtypeStruct((NC, 10, 1000), jnp.float32),
        mesh=_sc_mesh(),
        scratch_types=[
            pltpu.VMEM_SHARED((N,), jnp.float32),
            pltpu.VMEM((RPW, CH), jnp.int32),
            pltpu.VMEM((CH,), jnp.float32),
            pltpu.VMEM((1000,), jnp.float32),
            [pltpu.SemaphoreType.DMA for _ in range(4)],
        ],
    )(dst, jnp.ones((CH,), jnp.float32), jnp.zeros((1000,), jnp.float32))

    y, dis = pl.pallas_call(
        _scale_body,
        out_shape=(
            jax.ShapeDtypeStruct((N, D), jnp.float32),
            jax.ShapeDtypeStruct((N, 1), jnp.float32),
        ),
    )(x, deg_part.reshape(NC, N, 1))

    agg_part = pl.kernel(
        _agg_body,
        out_type=jax.ShapeDtypeStruct((NC, N, D), jnp.float32),
        mesh=_sc_mesh(),
        scratch_types=[
            pltpu.VMEM_SHARED((N, D), jnp.float32),
            [pltpu.VMEM((2, CH), jnp.int32) for _ in range(4)],
            [pltpu.VMEM((CH, D), jnp.float32) for _ in range(2)],
            [pltpu.SemaphoreType.DMA for _ in range(4)],
            [pltpu.SemaphoreType.DMA for _ in range(2)],
            [pltpu.SemaphoreType.DMA for _ in range(2)],
        ],
    )(y, edge_index, jnp.zeros((CH, D), jnp.float32))

    wcat = jnp.transpose(W, (1, 2, 0)).reshape(D, D * H)
    bcat = jnp.transpose(b).reshape(1, D * H)

    bm = 1000
    out4 = pl.pallas_call(
        _head_body,
        grid=(N // bm,),
        in_specs=[
            pl.BlockSpec((NC, bm, D), lambda i: (0, i, 0)),
            pl.BlockSpec((bm, D), lambda i: (i, 0)),
            pl.BlockSpec((bm, 1), lambda i: (i, 0)),
            pl.BlockSpec((D, D * H), lambda i: (0, 0)),
            pl.BlockSpec((1, D * H), lambda i: (0, 0)),
        ],
        out_specs=pl.BlockSpec((bm, D * H), lambda i: (i, 0)),
        out_shape=jax.ShapeDtypeStruct((N, D * H), jnp.float32),
    )(agg_part, y, dis, wcat, bcat)

    return out4.reshape(N, D, H)


# speedup vs baseline: 1.1919x; 1.1919x over previous
import jax
import jax.numpy as jnp
from jax import lax
from jax.experimental import pallas as pl
from jax.experimental.pallas import tpu as pltpu
from jax.experimental.pallas import tpu_sc as plsc

N = 10000
E = 320000
D = 128
H = 4

NC = 2
NS = 16
NW = NC * NS
CH = 128
TR = E // CH
RPW = 80


def _worker_rows(wid):
    nch = jnp.where(wid == NW - 1, TR - (NW - 1) * RPW, RPW)
    return wid * RPW, nch


def _ei_chunk(ei_hbm, row):
    return ei_hbm.at[pl.ds(0, 2), pl.ds(row * CH, CH)]


def _deg_body(dst_hbm, ones_hbm, zeros_hbm, out_hbm,
              deg_sp, dstv, onesv, zv, dsems):
    c = lax.axis_index("c")
    s = lax.axis_index("s")
    wid = c * NS + s
    base, nch = _worker_rows(wid)

    @pl.when(wid < NW - 1)
    def _():
        pltpu.sync_copy(dst_hbm.at[pl.ds(base, RPW)], dstv)

    @pl.when(wid == NW - 1)
    def _():
        pltpu.sync_copy(dst_hbm.at[pl.ds((NW - 1) * RPW, 20)],
                        dstv.at[pl.ds(0, 20)])
    pltpu.sync_copy(ones_hbm, onesv)
    @pl.when(s < 10)
    def _():
        pltpu.sync_copy(zeros_hbm, zv)
        pltpu.sync_copy(zv, deg_sp.at[pl.ds(s * 1000, 1000)])
    plsc.subcore_barrier()

    def body(g, carry):
        for u in range(4):
            j = g * 4 + u

            @pl.when(j >= 4)
            def _():
                pltpu.make_async_copy(onesv, deg_sp.at[dstv.at[0]],
                                      dsems[u]).wait()
            pltpu.async_copy(onesv, deg_sp.at[dstv.at[j]], dsems[u],
                             add=True)
        return carry

    lax.fori_loop(0, nch // 4, body, 0)
    for u in range(4):
        pltpu.make_async_copy(onesv, deg_sp.at[dstv.at[0]], dsems[u]).wait()
    plsc.subcore_barrier()

    @pl.when(s < 10)
    def _():
        pltpu.sync_copy(deg_sp.at[pl.ds(s * 1000, 1000)], zv)
        pltpu.sync_copy(zv, out_hbm.at[c, s])


def _agg_body(y_hbm, ei_hbm, zeros_hbm, out_hbm,
              agg_sp, sring, bufs, isems, gsems, ssems):
    c = lax.axis_index("c")
    s = lax.axis_index("s")
    wid = c * NS + s
    base, nch = _worker_rows(wid)
    pltpu.sync_copy(zeros_hbm, bufs[0])

    @pl.when(s < 15)
    def _():
        for k in range(5):
            pltpu.sync_copy(bufs[0],
                            agg_sp.at[pl.ds(s * 640 + k * 128, 128)])

    @pl.when(s == 15)
    def _():
        for k in range(3):
            pltpu.sync_copy(bufs[0],
                            agg_sp.at[pl.ds(9600 + k * 128, 128)])
        pltpu.sync_copy(bufs[0].at[pl.ds(0, 16)],
                        agg_sp.at[pl.ds(9984, 16)])
    plsc.subcore_barrier()

    for r in range(3):
        pltpu.async_copy(_ei_chunk(ei_hbm, base + r), sring[r], isems[r])
    pltpu.make_async_copy(_ei_chunk(ei_hbm, base), sring[0], isems[0]).wait()
    pltpu.async_copy(y_hbm.at[sring[0].at[0]], bufs[0], gsems[0])

    def group(g, carry):
        for u in range(4):
            j = g * 4 + u
            rn = (u + 1) % 4
            bn = (u + 1) % 2

            @pl.when(j + 1 < nch)
            def _():
                @pl.when(j >= 1)
                def _():
                    pltpu.make_async_copy(bufs[bn],
                                          agg_sp.at[sring[0].at[1]],
                                          ssems[bn]).wait()
                pltpu.make_async_copy(_ei_chunk(ei_hbm, base), sring[rn],
                                      isems[rn]).wait()
                pltpu.async_copy(y_hbm.at[sring[rn].at[0]], bufs[bn],
                                 gsems[bn])

            @pl.when(j + 3 < nch)
            def _():
                pltpu.async_copy(_ei_chunk(ei_hbm, base + j + 3),
                                 sring[(u + 3) % 4], isems[(u + 3) % 4])

            pltpu.make_async_copy(y_hbm.at[sring[u].at[0]], bufs[u % 2],
                                  gsems[u % 2]).wait()
            pltpu.async_copy(bufs[u % 2], agg_sp.at[sring[u].at[1]],
                             ssems[u % 2], add=True)
        return carry

    lax.fori_loop(0, nch // 4, group, 0)
    for b in range(2):
        pltpu.make_async_copy(bufs[b], agg_sp.at[sring[0].at[1]],
                              ssems[b]).wait()
    plsc.subcore_barrier()

    @pl.when(s < 15)
    def _():
        pltpu.sync_copy(agg_sp.at[pl.ds(s * 640, 640)],
                        out_hbm.at[c, pl.ds(s * 640, 640)])

    @pl.when(s == 15)
    def _():
        pltpu.sync_copy(agg_sp.at[pl.ds(9600, 400)],
                        out_hbm.at[c, pl.ds(9600, 400)])


def _scale_body(x_ref, dp_ref, y_ref, dis_ref):
    deg = dp_ref[0] + dp_ref[1] + 1.0
    dis = lax.rsqrt(deg)
    y_ref[...] = x_ref[...] * dis
    dis_ref[...] = dis


def _head_body(ap_ref, y_ref, dis_ref, w_ref, b_ref, o_ref):
    agg = (ap_ref[0] + ap_ref[1] + y_ref[...]) * dis_ref[...]
    for h in range(H):
        o_ref[h] = (
            jnp.dot(agg, w_ref[h], preferred_element_type=jnp.float32)
            + b_ref[h][None, :]
        )


def _sc_mesh():
    return plsc.VectorSubcoreMesh(core_axis_name="c", subcore_axis_name="s")


@jax.jit
def kernel(x, edge_index, W, b):
    dst = edge_index[1].reshape(TR, CH)
    deg_part = pl.kernel(
        _deg_body,
        out_type=jax.ShapeDtypeStruct((NC, 10, 1000), jnp.float32),
        mesh=_sc_mesh(),
        scratch_types=[
            pltpu.VMEM_SHARED((N,), jnp.float32),
            pltpu.VMEM((RPW, CH), jnp.int32),
            pltpu.VMEM((CH,), jnp.float32),
            pltpu.VMEM((1000,), jnp.float32),
            [pltpu.SemaphoreType.DMA for _ in range(4)],
        ],
    )(dst, jnp.ones((CH,), jnp.float32), jnp.zeros((1000,), jnp.float32))

    y, dis = pl.pallas_call(
        _scale_body,
        out_shape=(
            jax.ShapeDtypeStruct((N, D), jnp.float32),
            jax.ShapeDtypeStruct((N, 1), jnp.float32),
        ),
    )(x, deg_part.reshape(NC, N, 1))

    agg_part = pl.kernel(
        _agg_body,
        out_type=jax.ShapeDtypeStruct((NC, N, D), jnp.float32),
        mesh=_sc_mesh(),
        scratch_types=[
            pltpu.VMEM_SHARED((N, D), jnp.float32),
            [pltpu.VMEM((2, CH), jnp.int32) for _ in range(4)],
            [pltpu.VMEM((CH, D), jnp.float32) for _ in range(2)],
            [pltpu.SemaphoreType.DMA for _ in range(4)],
            [pltpu.SemaphoreType.DMA for _ in range(2)],
            [pltpu.SemaphoreType.DMA for _ in range(2)],
        ],
    )(y, edge_index, jnp.zeros((CH, D), jnp.float32))

    bm = 1000
    out4 = pl.pallas_call(
        _head_body,
        grid=(N // bm,),
        in_specs=[
            pl.BlockSpec((NC, bm, D), lambda i: (0, i, 0)),
            pl.BlockSpec((bm, D), lambda i: (i, 0)),
            pl.BlockSpec((bm, 1), lambda i: (i, 0)),
            pl.BlockSpec((H, D, D), lambda i: (0, 0, 0)),
            pl.BlockSpec((H, D), lambda i: (0, 0)),
        ],
        out_specs=pl.BlockSpec((H, bm, D), lambda i: (0, i, 0)),
        out_shape=jax.ShapeDtypeStruct((H, N, D), jnp.float32),
    )(agg_part, y, dis, W, b)

    return jnp.transpose(out4, (1, 2, 0))


# speedup vs baseline: 1.2793x; 1.0733x over previous
import jax
import jax.numpy as jnp
from jax import lax
from jax.experimental import pallas as pl
from jax.experimental.pallas import tpu as pltpu
from jax.experimental.pallas import tpu_sc as plsc

N = 10000
E = 320000
D = 128
H = 4

NC = 2
NS = 16
NW = NC * NS
CH = 128
TR = E // CH
RPW = 80


def _worker_rows(wid):
    nch = jnp.where(wid == NW - 1, TR - (NW - 1) * RPW, RPW)
    return wid * RPW, nch


def _ei_chunk(ei_hbm, row):
    return ei_hbm.at[pl.ds(0, 2), pl.ds(row * CH, CH)]


def _deg_body(ei_hbm, ones_hbm, zeros_hbm, out_hbm,
              deg_sp, ring, onesv, zv, isems, dsems):
    c = lax.axis_index("c")
    s = lax.axis_index("s")
    wid = c * NS + s
    base, nch = _worker_rows(wid)
    pltpu.sync_copy(ones_hbm, onesv)
    @pl.when(s < 10)
    def _():
        pltpu.sync_copy(zeros_hbm, zv)
        pltpu.sync_copy(zv, deg_sp.at[pl.ds(s * 1000, 1000)])
    plsc.subcore_barrier()

    for r in range(2):
        pltpu.async_copy(_ei_chunk(ei_hbm, base + r), ring[r], isems[r])

    def body(g, carry):
        for u in range(4):
            j = g * 4 + u

            @pl.when(j >= 2)
            def _():
                pltpu.make_async_copy(onesv, deg_sp.at[ring[0].at[1]],
                                      dsems[u % 2]).wait()

            @pl.when(j + 2 < nch)
            def _():
                pltpu.async_copy(_ei_chunk(ei_hbm, base + j + 2),
                                 ring[(u + 2) % 4], isems[(u + 2) % 4])
            pltpu.make_async_copy(_ei_chunk(ei_hbm, base), ring[u],
                                  isems[u]).wait()
            pltpu.async_copy(onesv, deg_sp.at[ring[u].at[1]], dsems[u % 2],
                             add=True)
        return carry

    lax.fori_loop(0, nch // 4, body, 0)
    for k in range(2):
        pltpu.make_async_copy(onesv, deg_sp.at[ring[0].at[1]],
                              dsems[k]).wait()
    plsc.subcore_barrier()

    @pl.when(s < 10)
    def _():
        pltpu.sync_copy(deg_sp.at[pl.ds(s * 1000, 1000)], zv)
        pltpu.sync_copy(zv, out_hbm.at[c, s])


def _agg_body(y_hbm, ei_hbm, zeros_hbm, out_hbm,
              agg_sp, sring, bufs, isems, gsems, ssems):
    c = lax.axis_index("c")
    s = lax.axis_index("s")
    wid = c * NS + s
    base, nch = _worker_rows(wid)
    pltpu.sync_copy(zeros_hbm, bufs[0])

    @pl.when(s < 15)
    def _():
        for k in range(5):
            pltpu.sync_copy(bufs[0],
                            agg_sp.at[pl.ds(s * 640 + k * 128, 128)])

    @pl.when(s == 15)
    def _():
        for k in range(3):
            pltpu.sync_copy(bufs[0],
                            agg_sp.at[pl.ds(9600 + k * 128, 128)])
        pltpu.sync_copy(bufs[0].at[pl.ds(0, 16)],
                        agg_sp.at[pl.ds(9984, 16)])
    plsc.subcore_barrier()

    for r in range(3):
        pltpu.async_copy(_ei_chunk(ei_hbm, base + r), sring[r], isems[r])
    pltpu.make_async_copy(_ei_chunk(ei_hbm, base), sring[0], isems[0]).wait()
    pltpu.async_copy(y_hbm.at[sring[0].at[0]], bufs[0], gsems[0])

    def group(g, carry):
        for u in range(4):
            j = g * 4 + u
            rn = (u + 1) % 4
            bn = (u + 1) % 2

            @pl.when(j + 1 < nch)
            def _():
                @pl.when(j >= 1)
                def _():
                    pltpu.make_async_copy(bufs[bn],
                                          agg_sp.at[sring[0].at[1]],
                                          ssems[bn]).wait()
                pltpu.make_async_copy(_ei_chunk(ei_hbm, base), sring[rn],
                                      isems[rn]).wait()
                pltpu.async_copy(y_hbm.at[sring[rn].at[0]], bufs[bn],
                                 gsems[bn])

            @pl.when(j + 3 < nch)
            def _():
                pltpu.async_copy(_ei_chunk(ei_hbm, base + j + 3),
                                 sring[(u + 3) % 4], isems[(u + 3) % 4])

            pltpu.make_async_copy(y_hbm.at[sring[u].at[0]], bufs[u % 2],
                                  gsems[u % 2]).wait()
            pltpu.async_copy(bufs[u % 2], agg_sp.at[sring[u].at[1]],
                             ssems[u % 2], add=True)
        return carry

    lax.fori_loop(0, nch // 4, group, 0)
    for b in range(2):
        pltpu.make_async_copy(bufs[b], agg_sp.at[sring[0].at[1]],
                              ssems[b]).wait()
    plsc.subcore_barrier()

    @pl.when(s < 15)
    def _():
        pltpu.sync_copy(agg_sp.at[pl.ds(s * 640, 640)],
                        out_hbm.at[c, pl.ds(s * 640, 640)])

    @pl.when(s == 15)
    def _():
        pltpu.sync_copy(agg_sp.at[pl.ds(9600, 400)],
                        out_hbm.at[c, pl.ds(9600, 400)])


def _scale_body(x_ref, dp_ref, y_ref, dis_ref):
    i = pl.program_id(0)
    deg = (dp_ref[0, pl.ds(i, 1), :] + dp_ref[1, pl.ds(i, 1), :]
           + 1.0)
    dis = lax.rsqrt(deg)
    y_ref[...] = x_ref[...] * jnp.transpose(dis, (1, 0))
    dis_ref[pl.ds(i, 1), :] = dis


def _head_body(ap_ref, y_ref, dis_ref, w_ref, b_ref, o_ref):
    i = pl.program_id(0)
    dis = jnp.transpose(dis_ref[pl.ds(i, 1), :], (1, 0))
    agg = (ap_ref[0] + ap_ref[1] + y_ref[...]) * dis
    for h in range(H):
        o_ref[h] = (
            jnp.dot(agg, w_ref[h], preferred_element_type=jnp.float32)
            + b_ref[h][None, :]
        )


def _sc_mesh():
    return plsc.VectorSubcoreMesh(core_axis_name="c", subcore_axis_name="s")


@jax.jit
def kernel(x, edge_index, W, b):
    deg_part = pl.kernel(
        _deg_body,
        out_type=jax.ShapeDtypeStruct((NC, 10, 1000), jnp.float32),
        mesh=_sc_mesh(),
        scratch_types=[
            pltpu.VMEM_SHARED((N,), jnp.float32),
            [pltpu.VMEM((2, CH), jnp.int32) for _ in range(4)],
            pltpu.VMEM((CH,), jnp.float32),
            pltpu.VMEM((1000,), jnp.float32),
            [pltpu.SemaphoreType.DMA for _ in range(4)],
            [pltpu.SemaphoreType.DMA for _ in range(2)],
        ],
    )(edge_index, jnp.ones((CH,), jnp.float32),
      jnp.zeros((1000,), jnp.float32))

    bm = 1000
    y, dis = pl.pallas_call(
        _scale_body,
        grid=(N // bm,),
        in_specs=[
            pl.BlockSpec((bm, D), lambda i: (i, 0)),
            pl.BlockSpec((NC, 10, bm), lambda i: (0, 0, 0)),
        ],
        out_specs=(
            pl.BlockSpec((bm, D), lambda i: (i, 0)),
            pl.BlockSpec((10, bm), lambda i: (0, 0)),
        ),
        out_shape=(
            jax.ShapeDtypeStruct((N, D), jnp.float32),
            jax.ShapeDtypeStruct((10, bm), jnp.float32),
        ),
    )(x, deg_part)

    agg_part = pl.kernel(
        _agg_body,
        out_type=jax.ShapeDtypeStruct((NC, N, D), jnp.float32),
        mesh=_sc_mesh(),
        scratch_types=[
            pltpu.VMEM_SHARED((N, D), jnp.float32),
            [pltpu.VMEM((2, CH), jnp.int32) for _ in range(4)],
            [pltpu.VMEM((CH, D), jnp.float32) for _ in range(2)],
            [pltpu.SemaphoreType.DMA for _ in range(4)],
            [pltpu.SemaphoreType.DMA for _ in range(2)],
            [pltpu.SemaphoreType.DMA for _ in range(2)],
        ],
    )(y, edge_index, jnp.zeros((CH, D), jnp.float32))

    out4 = pl.pallas_call(
        _head_body,
        grid=(N // bm,),
        in_specs=[
            pl.BlockSpec((NC, bm, D), lambda i: (0, i, 0)),
            pl.BlockSpec((bm, D), lambda i: (i, 0)),
            pl.BlockSpec((10, bm), lambda i: (0, 0)),
            pl.BlockSpec((H, D, D), lambda i: (0, 0, 0)),
            pl.BlockSpec((H, D), lambda i: (0, 0)),
        ],
        out_specs=pl.BlockSpec((H, bm, D), lambda i: (0, i, 0)),
        out_shape=jax.ShapeDtypeStruct((H, N, D), jnp.float32),
    )(agg_part, y, dis, W, b)

    return jnp.transpose(out4, (1, 2, 0))


# speedup vs baseline: 1.3423x; 1.0492x over previous
import jax
import jax.numpy as jnp
from jax import lax
from jax.experimental import pallas as pl
from jax.experimental.pallas import tpu as pltpu
from jax.experimental.pallas import tpu_sc as plsc

N = 10000
E = 320000
D = 128
H = 4

NC = 2
NS = 16
NW = NC * NS
CH = 128
TR = E // CH
RPW = 80


def _worker_rows(wid):
    nch = jnp.where(wid == NW - 1, TR - (NW - 1) * RPW, RPW)
    return wid * RPW, nch


def _ei_chunk(ei_hbm, row):
    return ei_hbm.at[pl.ds(0, 2), pl.ds(row * CH, CH)]


def _deg_body(ei_hbm, ones_hbm, zeros_hbm, out_hbm,
              deg_sp, ring, onesv, zv, isems, dsems):
    c = lax.axis_index("c")
    s = lax.axis_index("s")
    wid = c * NS + s
    base, nch = _worker_rows(wid)
    pltpu.sync_copy(ones_hbm, onesv)
    @pl.when(s < 10)
    def _():
        pltpu.sync_copy(zeros_hbm, zv)
        pltpu.sync_copy(zv, deg_sp.at[pl.ds(s * 1000, 1000)])
    plsc.subcore_barrier()

    nsl = nch // 5

    def _slot(t):
        return ei_hbm.at[pl.ds(0, 2), pl.ds((base + t * 5) * CH, 5 * CH)]

    for r in range(2):
        pltpu.async_copy(_slot(r), ring[r], isems[r])

    def body(g, carry):
        for u in range(4):
            t = g * 4 + u
            pltpu.make_async_copy(_slot(t), ring[u], isems[u]).wait()
            for k in range(5):
                @pl.when(t >= 1)
                def _():
                    pltpu.make_async_copy(
                        onesv, deg_sp.at[ring[0].at[1, pl.ds(0, CH)]],
                        dsems[k]).wait()
                pltpu.async_copy(onesv,
                                 deg_sp.at[ring[u].at[1, pl.ds(k * CH, CH)]],
                                 dsems[k], add=True)

            @pl.when(t + 2 < nsl)
            def _():
                pltpu.async_copy(_slot(t + 2), ring[(u + 2) % 4],
                                 isems[(u + 2) % 4])
        return carry

    lax.fori_loop(0, nsl // 4, body, 0)
    for k in range(5):
        pltpu.make_async_copy(onesv,
                              deg_sp.at[ring[0].at[1, pl.ds(0, CH)]],
                              dsems[k]).wait()
    plsc.subcore_barrier()

    @pl.when(s < 10)
    def _():
        pltpu.sync_copy(deg_sp.at[pl.ds(s * 1000, 1000)], zv)
        pltpu.sync_copy(zv, out_hbm.at[c, s])


def _agg_body(y_hbm, ei_hbm, zeros_hbm, out_hbm,
              agg_sp, sring, bufs, isems, gsems, ssems):
    c = lax.axis_index("c")
    s = lax.axis_index("s")
    wid = c * NS + s
    base, nch = _worker_rows(wid)
    pltpu.sync_copy(zeros_hbm, bufs[0])

    @pl.when(s < 15)
    def _():
        for k in range(5):
            pltpu.sync_copy(bufs[0],
                            agg_sp.at[pl.ds(s * 640 + k * 128, 128)])

    @pl.when(s == 15)
    def _():
        for k in range(3):
            pltpu.sync_copy(bufs[0],
                            agg_sp.at[pl.ds(9600 + k * 128, 128)])
        pltpu.sync_copy(bufs[0].at[pl.ds(0, 16)],
                        agg_sp.at[pl.ds(9984, 16)])
    plsc.subcore_barrier()

    for r in range(3):
        pltpu.async_copy(_ei_chunk(ei_hbm, base + r), sring[r], isems[r])
    pltpu.make_async_copy(_ei_chunk(ei_hbm, base), sring[0], isems[0]).wait()
    pltpu.async_copy(y_hbm.at[sring[0].at[0]], bufs[0], gsems[0])

    def group(g, carry):
        for u in range(4):
            j = g * 4 + u
            rn = (u + 1) % 4
            bn = (u + 1) % 2

            @pl.when(j + 1 < nch)
            def _():
                @pl.when(j >= 1)
                def _():
                    pltpu.make_async_copy(bufs[bn],
                                          agg_sp.at[sring[0].at[1]],
                                          ssems[bn]).wait()
                pltpu.make_async_copy(_ei_chunk(ei_hbm, base), sring[rn],
                                      isems[rn]).wait()
                pltpu.async_copy(y_hbm.at[sring[rn].at[0]], bufs[bn],
                                 gsems[bn])

            @pl.when(j + 3 < nch)
            def _():
                pltpu.async_copy(_ei_chunk(ei_hbm, base + j + 3),
                                 sring[(u + 3) % 4], isems[(u + 3) % 4])

            pltpu.make_async_copy(y_hbm.at[sring[u].at[0]], bufs[u % 2],
                                  gsems[u % 2]).wait()
            pltpu.async_copy(bufs[u % 2], agg_sp.at[sring[u].at[1]],
                             ssems[u % 2], add=True)
        return carry

    lax.fori_loop(0, nch // 4, group, 0)
    for b in range(2):
        pltpu.make_async_copy(bufs[b], agg_sp.at[sring[0].at[1]],
                              ssems[b]).wait()
    plsc.subcore_barrier()

    @pl.when(s < 15)
    def _():
        pltpu.sync_copy(agg_sp.at[pl.ds(s * 640, 640)],
                        out_hbm.at[c, pl.ds(s * 640, 640)])

    @pl.when(s == 15)
    def _():
        pltpu.sync_copy(agg_sp.at[pl.ds(9600, 400)],
                        out_hbm.at[c, pl.ds(9600, 400)])


def _scale_body(x_ref, dp_ref, y_ref, dis_ref):
    i = pl.program_id(0)
    deg = (dp_ref[0, pl.ds(i, 1), :] + dp_ref[1, pl.ds(i, 1), :]
           + 1.0)
    dis = lax.rsqrt(deg)
    y_ref[...] = x_ref[...] * jnp.transpose(dis, (1, 0))
    dis_ref[pl.ds(i, 1), :] = dis


def _head_body(ap_ref, y_ref, dis_ref, w_ref, b_ref, o_ref):
    i = pl.program_id(0)
    dis = jnp.transpose(dis_ref[pl.ds(i, 1), :], (1, 0))
    agg = (ap_ref[0] + ap_ref[1] + y_ref[...]) * dis
    for h in range(H):
        o_ref[h] = (
            jnp.dot(agg, w_ref[h], preferred_element_type=jnp.float32)
            + b_ref[h][None, :]
        )


def _sc_mesh():
    return plsc.VectorSubcoreMesh(core_axis_name="c", subcore_axis_name="s")


@jax.jit
def kernel(x, edge_index, W, b):
    deg_part = pl.kernel(
        _deg_body,
        out_type=jax.ShapeDtypeStruct((NC, 10, 1000), jnp.float32),
        mesh=_sc_mesh(),
        scratch_types=[
            pltpu.VMEM_SHARED((N,), jnp.float32),
            [pltpu.VMEM((2, 5 * CH), jnp.int32) for _ in range(4)],
            pltpu.VMEM((CH,), jnp.float32),
            pltpu.VMEM((1000,), jnp.float32),
            [pltpu.SemaphoreType.DMA for _ in range(4)],
            [pltpu.SemaphoreType.DMA for _ in range(5)],
        ],
    )(edge_index, jnp.ones((CH,), jnp.float32),
      jnp.zeros((1000,), jnp.float32))

    bm = 1000
    y, dis = pl.pallas_call(
        _scale_body,
        grid=(N // bm,),
        in_specs=[
            pl.BlockSpec((bm, D), lambda i: (i, 0)),
            pl.BlockSpec((NC, 10, bm), lambda i: (0, 0, 0)),
        ],
        out_specs=(
            pl.BlockSpec((bm, D), lambda i: (i, 0)),
            pl.BlockSpec((10, bm), lambda i: (0, 0)),
        ),
        out_shape=(
            jax.ShapeDtypeStruct((N, D), jnp.float32),
            jax.ShapeDtypeStruct((10, bm), jnp.float32),
        ),
    )(x, deg_part)

    agg_part = pl.kernel(
        _agg_body,
        out_type=jax.ShapeDtypeStruct((NC, N, D), jnp.float32),
        mesh=_sc_mesh(),
        scratch_types=[
            pltpu.VMEM_SHARED((N, D), jnp.float32),
            [pltpu.VMEM((2, CH), jnp.int32) for _ in range(4)],
            [pltpu.VMEM((CH, D), jnp.float32) for _ in range(2)],
            [pltpu.SemaphoreType.DMA for _ in range(4)],
            [pltpu.SemaphoreType.DMA for _ in range(2)],
            [pltpu.SemaphoreType.DMA for _ in range(2)],
        ],
    )(y, edge_index, jnp.zeros((CH, D), jnp.float32))

    out4 = pl.pallas_call(
        _head_body,
        grid=(N // bm,),
        in_specs=[
            pl.BlockSpec((NC, bm, D), lambda i: (0, i, 0)),
            pl.BlockSpec((bm, D), lambda i: (i, 0)),
            pl.BlockSpec((10, bm), lambda i: (0, 0)),
            pl.BlockSpec((H, D, D), lambda i: (0, 0, 0)),
            pl.BlockSpec((H, D), lambda i: (0, 0)),
        ],
        out_specs=pl.BlockSpec((H, bm, D), lambda i: (0, i, 0)),
        out_shape=jax.ShapeDtypeStruct((H, N, D), jnp.float32),
    )(agg_part, y, dis, W, b)

    return jnp.transpose(out4, (1, 2, 0))
